# Initial kernel scaffold; baseline (speedup 1.0000x reference)
import jax
import jax.numpy as jnp
from jax.experimental import pallas as pl


def kernel(user, items, ids, reward, user_embedding, item_embedding):
    raise NotImplementedError("write your pallas kernel here")


# speedup vs baseline: 1.3072x; 1.3072x over previous
import functools

import jax
import jax.numpy as jnp
from jax import lax
from jax.experimental import pallas as pl
from jax.experimental.pallas import tpu as pltpu
from jax.experimental.pallas import tpu_sc as plsc

NCORES = 2
NSUB = 16
NW = NCORES * NSUB
LANES = 16

B = 4096
L = 200
E = 32
LP = 208
NGROUPS = LP // LANES
RPW = B // NW
NBUF = 2
REGS = 1e-05


def _issue_row_gather(iemb_hbm, iidx_v, slot, sem, r):
  pltpu.async_copy(iemb_hbm.at[iidx_v.at[pl.ds(r * L, 104)]],
                   slot.at[pl.ds(0, 104), :], sem)
  pltpu.async_copy(iemb_hbm.at[iidx_v.at[pl.ds(r * L + 96, 104)]],
                   slot.at[pl.ds(96, 104), :], sem)


def _drain_row_gather(iemb_hbm, slot, sem):
  pltpu.make_async_copy(iemb_hbm.at[pl.ds(0, LP), :], slot, sem).wait()


def _sc_body(user_hbm, items_hbm, ids_hbm, uemb_hbm, iemb_hbm,
             sel_hbm, sumexp_hbm, ss_hbm,
             uidx_v, iidx_v, ids_v, urows_v, rows0_v, rows1_v,
             sel_v, sumexp_v, ss_stage_v, sem0, sem1, semu):
  c = lax.axis_index("c")
  s = lax.axis_index("s")
  wid = s * NCORES + c
  base = wid * RPW

  pltpu.sync_copy(user_hbm.at[pl.ds(base, RPW)], uidx_v)
  pltpu.sync_copy(ids_hbm.at[pl.ds(base, RPW)], ids_v)
  pltpu.sync_copy(items_hbm.at[pl.ds(base * L, RPW * L)], iidx_v)

  ucopy = pltpu.async_copy(uemb_hbm.at[uidx_v], urows_v, semu)

  zero = jnp.zeros((LANES,), jnp.float32)
  for slot in (rows0_v, rows1_v):
    for r in range(L, LP):
      slot[r, pl.ds(0, LANES)] = zero
      slot[r, pl.ds(LANES, LANES)] = zero

  _issue_row_gather(iemb_hbm, iidx_v, rows0_v, sem0, 0)
  _issue_row_gather(iemb_hbm, iidx_v, rows1_v, sem1, 1)
  ucopy.wait()

  lane = lax.iota(jnp.int32, LANES)

  def compute_row(r, slot, ssv):
    idb = plsc.load_gather(ids_v, [jnp.full((LANES,), r, jnp.int32)])
    uv0 = urows_v[r, pl.ds(0, LANES)]
    uv1 = urows_v[r, pl.ds(LANES, LANES)]
    ue = [uv0[e] for e in range(LANES)] + [uv1[e] for e in range(LANES)]
    sumexp_vec = jnp.zeros((LANES,), jnp.float32)
    sel_vec = jnp.zeros((LANES,), jnp.float32)
    for g in range(NGROUPS):
      item_idx = lane + g * LANES
      acc = jnp.zeros((LANES,), jnp.float32)
      for e in range(E):
        v = plsc.load_gather(slot, [item_idx, jnp.full((LANES,), e, jnp.int32)])
        acc = acc + v * ue[e]
        ssv = ssv + v * v
      ex = jnp.exp(acc)
      if g == NGROUPS - 1:
        ex = jnp.where(lane < L - (NGROUPS - 1) * LANES, ex, 0.0)
      sumexp_vec = sumexp_vec + ex
      sel_vec = sel_vec + jnp.where(item_idx == idb, acc, 0.0)
    ssv = ssv + uv0 * uv0 + uv1 * uv1
    sumexp_v[r, :] = sumexp_vec
    sel_v[r, :] = sel_vec
    return ssv

  def body(i, ssv):
    r0 = i * NBUF
    for j, (slot, sem) in enumerate(((rows0_v, sem0), (rows1_v, sem1))):
      r = r0 + j
      _drain_row_gather(iemb_hbm, slot, sem)
      ssv = compute_row(r, slot, ssv)
      rn = r + NBUF
      @pl.when(rn < RPW)
      def _():
        _issue_row_gather(iemb_hbm, iidx_v, slot, sem, rn)
    return ssv

  ssv = lax.fori_loop(0, RPW // NBUF, body, jnp.zeros((LANES,), jnp.float32))

  ss_stage_v[...] = ssv
  pltpu.sync_copy(sel_v, sel_hbm.at[pl.ds(base, RPW), :])
  pltpu.sync_copy(sumexp_v, sumexp_hbm.at[pl.ds(base, RPW), :])
  pltpu.sync_copy(ss_stage_v, ss_hbm.at[pl.ds(wid * LANES, LANES)])


def _tc_finish(sel_ref, sume_ref, rew_ref, ss_ref, gan_ref, reg_ref):
  sel = jnp.sum(sel_ref[...], axis=1, keepdims=True)
  se = jnp.sum(sume_ref[...], axis=1, keepdims=True)
  t = (sel - jnp.log(se)) * rew_ref[...]
  gan_ref[...] = jnp.reshape(-jnp.sum(t) / float(B), (1, 1))
  reg_ref[...] = jnp.reshape(jnp.float32(REGS * 0.5) * jnp.sum(ss_ref[...]),
                             (1, 1))


def kernel(user, items, ids, reward, user_embedding, item_embedding):
  user = user.astype(jnp.int32)
  items = items.reshape(B * L).astype(jnp.int32)
  ids_flat = ids.reshape(B).astype(jnp.int32)

  sc = functools.partial(
      pl.kernel,
      out_type=[
          jax.ShapeDtypeStruct((B, LANES), jnp.float32),
          jax.ShapeDtypeStruct((B, LANES), jnp.float32),
          jax.ShapeDtypeStruct((NW * LANES,), jnp.float32),
      ],
      scratch_types=[
          pltpu.VMEM((RPW,), jnp.int32),
          pltpu.VMEM((RPW * L,), jnp.int32),
          pltpu.VMEM((RPW,), jnp.int32),
          pltpu.VMEM((RPW, E), jnp.float32),
          pltpu.VMEM((LP, E), jnp.float32),
          pltpu.VMEM((LP, E), jnp.float32),
          pltpu.VMEM((RPW, LANES), jnp.float32),
          pltpu.VMEM((RPW, LANES), jnp.float32),
          pltpu.VMEM((LANES,), jnp.float32),
          pltpu.SemaphoreType.DMA,
          pltpu.SemaphoreType.DMA,
          pltpu.SemaphoreType.DMA,
      ],
      mesh=plsc.VectorSubcoreMesh(core_axis_name="c", subcore_axis_name="s"),
      compiler_params=pltpu.CompilerParams(needs_layout_passes=False,
                                           use_tc_tiling_on_sc=False),
  )(_sc_body)

  sel, sumexp, ss = sc(user, items, ids_flat, user_embedding, item_embedding)

  gan, reg = pl.pallas_call(
      _tc_finish,
      out_shape=[
          jax.ShapeDtypeStruct((1, 1), jnp.float32),
          jax.ShapeDtypeStruct((1, 1), jnp.float32),
      ],
  )(sel, sumexp, reward.reshape(B, 1), ss.reshape(4, 128))

  return (gan[0, 0], reg[0, 0])


# speedup vs baseline: 1.4072x; 1.0765x over previous
import functools

import jax
import jax.numpy as jnp
from jax import lax
from jax.experimental import pallas as pl
from jax.experimental.pallas import tpu as pltpu
from jax.experimental.pallas import tpu_sc as plsc

NCORES = 2
NSUB = 16
NW = NCORES * NSUB
LANES = 16

B = 4096
L = 200
E = 32
W = 128
LP = 208
NGROUPS = LP // LANES
RPW = B // NW
NBUF = 2
REGS = 1e-05


def _issue_row_gather(iemb_hbm, iidx_v, slot, sem, r):
  for g in range(NGROUPS):
    iv = iidx_v[pl.ds(r * L + g * LANES, LANES)] >> 2
    pltpu.async_copy(iemb_hbm.at[iv],
                     slot.at[pl.ds(g * LANES, LANES), :], sem)


def _drain_row_gather(iemb_hbm, slot, sem):
  pltpu.make_async_copy(iemb_hbm.at[pl.ds(0, LP), :], slot, sem).wait()


def _sc_body(user_hbm, items_hbm, ids_hbm, uemb_hbm, iemb_hbm,
             sel_hbm, sumexp_hbm, ss_hbm,
             uidx_v, upidx_v, iidx_v, ids_v, urows_v,
             rows0_v, rows1_v, sel_v, sumexp_v, ss_stage_v,
             sem0, sem1, semu):
  c = lax.axis_index("c")
  s = lax.axis_index("s")
  wid = s * NCORES + c
  base = wid * RPW

  pltpu.sync_copy(user_hbm.at[pl.ds(base, RPW)], uidx_v)
  pltpu.sync_copy(ids_hbm.at[pl.ds(base, RPW)], ids_v)
  pltpu.sync_copy(items_hbm.at[pl.ds(base * L, RPW * L)],
                  iidx_v.at[pl.ds(0, RPW * L)])

  iidx_v[pl.ds(RPW * L, LANES)] = jnp.zeros((LANES,), jnp.int32)

  for k in range(RPW // LANES):
    upidx_v[pl.ds(k * LANES, LANES)] = (
        uidx_v[pl.ds(k * LANES, LANES)] >> 2)

  ucopy = pltpu.async_copy(uemb_hbm.at[upidx_v], urows_v, semu)

  _issue_row_gather(iemb_hbm, iidx_v, rows0_v, sem0, 0)
  _issue_row_gather(iemb_hbm, iidx_v, rows1_v, sem1, 1)
  ucopy.wait()

  lane = lax.iota(jnp.int32, LANES)
  group_lanes = [lane + g * LANES for g in range(NGROUPS)]
  last_mask = lane < (L - (NGROUPS - 1) * LANES)
  tail_mask = lane == (LANES - 1)

  def compute_row(r, slot, ssv):
    rr = jnp.full((LANES,), r, jnp.int32)
    idb = plsc.load_gather(ids_v, [rr])
    uoff = (plsc.load_gather(uidx_v, [rr]) & 3) * E
    uv0 = plsc.load_gather(urows_v, [rr, uoff + lane])
    uv1 = plsc.load_gather(urows_v, [rr, uoff + lane + LANES])
    sumexp_vec = jnp.zeros((LANES,), jnp.float32)
    sel_vec = jnp.zeros((LANES,), jnp.float32)
    for g in range(NGROUPS):
      off = (iidx_v[pl.ds(r * L + g * LANES, LANES)] & 3) * E
      acc0 = jnp.zeros((LANES,), jnp.float32)
      acc1 = jnp.zeros((LANES,), jnp.float32)
      ss0 = jnp.zeros((LANES,), jnp.float32)
      ss1 = jnp.zeros((LANES,), jnp.float32)
      for e in range(E):
        ue = uv0[e] if e < LANES else uv1[e - LANES]
        v = plsc.load_gather(slot, [group_lanes[g], off + e])
        if g == NGROUPS - 1:
          v = jnp.where(last_mask, v, 0.0)
        if e % 2 == 0:
          acc0 = acc0 + v * ue
          ss0 = ss0 + v * v
        else:
          acc1 = acc1 + v * ue
          ss1 = ss1 + v * v
      acc = acc0 + acc1
      ssv = ssv + ss0 + ss1
      ex = jnp.exp(acc)
      if g == NGROUPS - 1:
        ex = jnp.where(last_mask, ex, 0.0)
      sumexp_vec = sumexp_vec + ex
      sel_vec = sel_vec + jnp.where(group_lanes[g] == idb, acc, 0.0)
    ssv = ssv + uv0 * uv0 + uv1 * uv1
    plsc.store_scatter(sumexp_v, [rr], jnp.cumsum(sumexp_vec),
                       mask=tail_mask)
    plsc.store_scatter(sel_v, [rr], jnp.cumsum(sel_vec), mask=tail_mask)
    return ssv

  def body(i, ssv):
    r0 = i * NBUF
    for j, (slot, sem) in enumerate(((rows0_v, sem0), (rows1_v, sem1))):
      r = r0 + j
      _drain_row_gather(iemb_hbm, slot, sem)
      ssv = compute_row(r, slot, ssv)
      rn = r + NBUF
      @pl.when(rn < RPW)
      def _():
        _issue_row_gather(iemb_hbm, iidx_v, slot, sem, rn)
    return ssv

  ssv = lax.fori_loop(0, RPW // NBUF, body, jnp.zeros((LANES,), jnp.float32))

  ss_stage_v[...] = ssv
  pltpu.sync_copy(sel_v, sel_hbm.at[pl.ds(base, RPW)])
  pltpu.sync_copy(sumexp_v, sumexp_hbm.at[pl.ds(base, RPW)])
  pltpu.sync_copy(ss_stage_v, ss_hbm.at[pl.ds(wid * LANES, LANES)])


def _tc_finish(sel_ref, sume_ref, rew_ref, ss_ref, gan_ref, reg_ref):
  t = (sel_ref[...] - jnp.log(sume_ref[...])) * rew_ref[...]
  gan_ref[...] = jnp.reshape(-jnp.sum(t) / float(B), (1, 1))
  reg_ref[...] = jnp.reshape(jnp.float32(REGS * 0.5) * jnp.sum(ss_ref[...]),
                             (1, 1))


def kernel(user, items, ids, reward, user_embedding, item_embedding):
  user = user.astype(jnp.int32)
  items = items.reshape(B * L).astype(jnp.int32)
  ids_flat = ids.reshape(B).astype(jnp.int32)
  uemb_w = user_embedding.reshape(-1, W)
  iemb_w = item_embedding.reshape(-1, W)

  sc = functools.partial(
      pl.kernel,
      out_type=[
          jax.ShapeDtypeStruct((B,), jnp.float32),
          jax.ShapeDtypeStruct((B,), jnp.float32),
          jax.ShapeDtypeStruct((NW * LANES,), jnp.float32),
      ],
      scratch_types=[
          pltpu.VMEM((RPW,), jnp.int32),
          pltpu.VMEM((RPW,), jnp.int32),
          pltpu.VMEM((RPW * L + LANES,), jnp.int32),
          pltpu.VMEM((RPW,), jnp.int32),
          pltpu.VMEM((RPW, W), jnp.float32),
          pltpu.VMEM((LP, W), jnp.float32),
          pltpu.VMEM((LP, W), jnp.float32),
          pltpu.VMEM((RPW,), jnp.float32),
          pltpu.VMEM((RPW,), jnp.float32),
          pltpu.VMEM((LANES,), jnp.float32),
          pltpu.SemaphoreType.DMA,
          pltpu.SemaphoreType.DMA,
          pltpu.SemaphoreType.DMA,
      ],
      mesh=plsc.VectorSubcoreMesh(core_axis_name="c", subcore_axis_name="s"),
      compiler_params=pltpu.CompilerParams(needs_layout_passes=False,
                                           use_tc_tiling_on_sc=True),
  )(_sc_body)

  sel, sumexp, ss = sc(user, items, ids_flat, uemb_w, iemb_w)

  gan, reg = pl.pallas_call(
      _tc_finish,
      out_shape=[
          jax.ShapeDtypeStruct((1, 1), jnp.float32),
          jax.ShapeDtypeStruct((1, 1), jnp.float32),
      ],
  )(sel.reshape(32, 128), sumexp.reshape(32, 128),
    reward.reshape(32, 128), ss.reshape(4, 128))

  return (gan[0, 0], reg[0, 0])
